# baseline (speedup 1.0000x reference)
import functools

import jax
import jax.numpy as jnp
from jax import lax
from jax.experimental import pallas as pl
from jax.experimental.pallas import tpu as pltpu
from jax.experimental.pallas import tpu_sc as plsc

_B, _S, _D, _E, _K = 4, 2048, 4096, 64, 8
_ALPHA = 0.1
_RATIO = 0.9
_DEAD = 0.01
_TOK_BLOCK = 1024
_N_TOK = _B * _S
_NW = 32
_TPW = _N_TOK // _NW
_L = 16


def _dense_body(h_ref, wct_ref, scores_ref, raw_ref):
    h = h_ref[...]
    wct = wct_ref[...]
    logits = jax.lax.dot_general(
        wct, h, (((1,), (1,)), ((), ())), preferred_element_type=jnp.float32
    )
    raw = logits[:_E, :]
    wm = logits[_E:, :]

    m = jnp.max(raw, axis=0, keepdims=True)
    e = jnp.exp(raw - m)
    probs = e / jnp.sum(e, axis=0, keepdims=True)

    iota = jax.lax.broadcasted_iota(jnp.int32, probs.shape, 0)

    p1 = jnp.max(probs, axis=0, keepdims=True)
    idx1 = jnp.min(jnp.where(probs == p1, iota, _E), axis=0, keepdims=True)
    p2 = jnp.max(jnp.where(iota == idx1, -jnp.inf, probs), axis=0, keepdims=True)
    safe = (p2 / (p1 + 1e-9)) >= _RATIO

    mean_raw = jnp.mean(raw, axis=0, keepdims=True)
    var_raw = jnp.sum((raw - mean_raw) ** 2, axis=0, keepdims=True) / (_E - 1)
    mean_wm = jnp.mean(wm, axis=0, keepdims=True)
    var_wm = jnp.sum((wm - mean_wm) ** 2, axis=0, keepdims=True) / (_E - 1)
    scale = jnp.sqrt(var_raw) / (jnp.sqrt(var_wm) + 1e-9)

    combined = safe & (jnp.abs(wm) >= _DEAD)
    scores_ref[...] = jnp.where(combined, raw + _ALPHA * wm * scale, raw)
    raw_ref[...] = raw


def _dense_call(h2, wct):
    grid = (_N_TOK // _TOK_BLOCK,)
    return pl.pallas_call(
        _dense_body,
        grid=grid,
        in_specs=[
            pl.BlockSpec((_TOK_BLOCK, _D), lambda i: (i, 0)),
            pl.BlockSpec((2 * _E, _D), lambda i: (0, 0)),
        ],
        out_specs=[
            pl.BlockSpec((_E, _TOK_BLOCK), lambda i: (0, i)),
            pl.BlockSpec((_E, _TOK_BLOCK), lambda i: (0, i)),
        ],
        out_shape=[
            jax.ShapeDtypeStruct((_E, _N_TOK), jnp.float32),
            jax.ShapeDtypeStruct((_E, _N_TOK), jnp.float32),
        ],
    )(h2, wct)


def _sc_router_body(scores_hbm, raw_hbm, w_hbm, sel_hbm, sv, rv, wv, iv):
    wid = lax.axis_index("s") * 2 + lax.axis_index("c")
    base = wid * _TPW
    pltpu.sync_copy(scores_hbm.at[:, pl.ds(base, _TPW)], sv)
    pltpu.sync_copy(raw_hbm.at[:, pl.ds(base, _TPW)], rv)

    col_iota = lax.iota(jnp.int32, _L)
    neg_inf = jnp.full((_L,), -jnp.inf, dtype=jnp.float32)
    zeros_i = jnp.zeros((_L,), dtype=jnp.int32)

    def group(g, _):
        g16 = g * _L
        r = [neg_inf] * _K
        idx = [zeros_i] * _K
        for e in range(_E):
            v = sv[e, pl.ds(g16, _L)]
            ei = jnp.full((_L,), e, dtype=jnp.int32)
            c = [v > r[j] for j in range(_K)]
            for j in range(_K - 1, 0, -1):
                shifted_r = jnp.where(c[j - 1], r[j - 1], v)
                shifted_i = jnp.where(c[j - 1], idx[j - 1], ei)
                r[j] = jnp.where(c[j], shifted_r, r[j])
                idx[j] = jnp.where(c[j], shifted_i, idx[j])
            r[0] = jnp.where(c[0], v, r[0])
            idx[0] = jnp.where(c[0], ei, idx[0])

        a = [plsc.load_gather(rv, [idx[j], g16 + col_iota]) for j in range(_K)]
        mx = a[0]
        for j in range(1, _K):
            mx = jnp.maximum(mx, a[j])
        ex = [jnp.exp(a[j] - mx) for j in range(_K)]
        z = ex[0]
        for j in range(1, _K):
            z = z + ex[j]
        for j in range(_K):
            wv[j, pl.ds(g16, _L)] = ex[j] / z
            iv[j, pl.ds(g16, _L)] = idx[j]
        return 0

    lax.fori_loop(0, _TPW // _L, group, 0)

    pltpu.sync_copy(wv, w_hbm.at[:, pl.ds(base, _TPW)])
    pltpu.sync_copy(iv, sel_hbm.at[:, pl.ds(base, _TPW)])


@functools.partial(
    pl.kernel,
    mesh=plsc.VectorSubcoreMesh(core_axis_name="c", subcore_axis_name="s"),
    compiler_params=pltpu.CompilerParams(use_tc_tiling_on_sc=False, needs_layout_passes=False),
    out_type=[
        jax.ShapeDtypeStruct((_K, _N_TOK), jnp.float32),
        jax.ShapeDtypeStruct((_K, _N_TOK), jnp.int32),
    ],
    scratch_types=[
        pltpu.VMEM((_E, _TPW), jnp.float32),
        pltpu.VMEM((_E, _TPW), jnp.float32),
        pltpu.VMEM((_K, _TPW), jnp.float32),
        pltpu.VMEM((_K, _TPW), jnp.int32),
    ],
)
def _sc_router(scores_hbm, raw_hbm, w_hbm, sel_hbm, sv, rv, wv, iv):
    _sc_router_body(scores_hbm, raw_hbm, w_hbm, sel_hbm, sv, rv, wv, iv)


@jax.jit
def kernel(hidden_states, W_gate, secret_projection):
    h2 = hidden_states.reshape(_N_TOK, _D)
    wct = jnp.concatenate([W_gate, secret_projection.T], axis=0)
    scores_t, raw_t = _dense_call(h2, wct)
    weights_t, experts_t = _sc_router(scores_t, raw_t)
    return (
        weights_t.T.reshape(_B, _S, _K),
        experts_t.T.reshape(_B, _S, _K),
    )


# speedup vs baseline: 1.4755x; 1.4755x over previous
import jax
import jax.numpy as jnp
from jax.experimental import pallas as pl

_B, _S, _D, _E, _K = 4, 2048, 4096, 64, 8
_ALPHA = 0.1
_RATIO = 0.9
_DEAD = 0.01
_TOK_BLOCK = 1024


def _router_body(h_ref, wg_ref, sp_ref, weights_ref, experts_ref):
    h = h_ref[...]
    raw = jax.lax.dot_general(
        wg_ref[...], h, (((1,), (1,)), ((), ())), preferred_element_type=jnp.float32
    )
    wm = jax.lax.dot_general(
        sp_ref[...], h, (((0,), (1,)), ((), ())), preferred_element_type=jnp.float32
    )

    m = jnp.max(raw, axis=0, keepdims=True)
    e = jnp.exp(raw - m)
    probs = e / jnp.sum(e, axis=0, keepdims=True)

    iota = jax.lax.broadcasted_iota(jnp.int32, probs.shape, 0)

    p1 = jnp.max(probs, axis=0, keepdims=True)
    idx1 = jnp.min(jnp.where(probs == p1, iota, _E), axis=0, keepdims=True)
    p2 = jnp.max(jnp.where(iota == idx1, -jnp.inf, probs), axis=0, keepdims=True)
    safe = (p2 / (p1 + 1e-9)) >= _RATIO

    mean_raw = jnp.mean(raw, axis=0, keepdims=True)
    var_raw = jnp.sum((raw - mean_raw) ** 2, axis=0, keepdims=True) / (_E - 1)
    mean_wm = jnp.mean(wm, axis=0, keepdims=True)
    var_wm = jnp.sum((wm - mean_wm) ** 2, axis=0, keepdims=True) / (_E - 1)
    scale = jnp.sqrt(var_raw) / (jnp.sqrt(var_wm) + 1e-9)

    combined = safe & (jnp.abs(wm) >= _DEAD)
    scores = jnp.where(combined, raw + _ALPHA * wm * scale, raw)

    idx_rows = []
    val_rows = []
    for _ in range(_K):
        mk = jnp.max(scores, axis=0, keepdims=True)
        idx = jnp.min(jnp.where(scores == mk, iota, _E), axis=0, keepdims=True)
        onehot = iota == idx
        val = jnp.sum(jnp.where(onehot, raw, 0.0), axis=0, keepdims=True)
        idx_rows.append(idx)
        val_rows.append(val)
        scores = jnp.where(onehot, -jnp.inf, scores)
    sel = jnp.concatenate(idx_rows, axis=0)
    vals = jnp.concatenate(val_rows, axis=0)

    mv = jnp.max(vals, axis=0, keepdims=True)
    ev = jnp.exp(vals - mv)
    weights_ref[...] = ev / jnp.sum(ev, axis=0, keepdims=True)
    experts_ref[...] = sel


@jax.jit
def kernel(hidden_states, W_gate, secret_projection):
    n_tok = _B * _S
    h2 = hidden_states.reshape(n_tok, _D)

    grid = (n_tok // _TOK_BLOCK,)
    weights_t, experts_t = pl.pallas_call(
        _router_body,
        grid=grid,
        in_specs=[
            pl.BlockSpec((_TOK_BLOCK, _D), lambda i: (i, 0)),
            pl.BlockSpec((_E, _D), lambda i: (0, 0)),
            pl.BlockSpec((_D, _E), lambda i: (0, 0)),
        ],
        out_specs=[
            pl.BlockSpec((_K, _TOK_BLOCK), lambda i: (0, i)),
            pl.BlockSpec((_K, _TOK_BLOCK), lambda i: (0, i)),
        ],
        out_shape=[
            jax.ShapeDtypeStruct((_K, n_tok), jnp.float32),
            jax.ShapeDtypeStruct((_K, n_tok), jnp.int32),
        ],
    )(h2, W_gate, secret_projection)
    return (
        weights_t.T.reshape(_B, _S, _K),
        experts_t.T.reshape(_B, _S, _K),
    )


# speedup vs baseline: 1.5737x; 1.0665x over previous
import jax
import jax.numpy as jnp
from jax.experimental import pallas as pl
from jax.experimental.pallas import tpu as pltpu

_B, _S, _D, _E, _K = 4, 2048, 4096, 64, 8
_ALPHA = 0.1
_RATIO = 0.9
_DEAD = 0.01
_TOK_BLOCK = 1024


def _router_body(h_ref, wct_hbm, weights_ref, experts_ref, wct_vmem, sem):
    @pl.when(pl.program_id(0) == 0)
    def _():
        pltpu.make_async_copy(wct_hbm, wct_vmem, sem).start()
        pltpu.make_async_copy(wct_hbm, wct_vmem, sem).wait()

    h = h_ref[...]
    wct = wct_vmem[...]
    logits = jax.lax.dot_general(
        wct, h, (((1,), (1,)), ((), ())), preferred_element_type=jnp.float32
    )
    raw = logits[:_E, :]
    wm = logits[_E:, :]

    m = jnp.max(raw, axis=0, keepdims=True)
    e = jnp.exp(raw - m)
    probs = e / jnp.sum(e, axis=0, keepdims=True)

    iota = jax.lax.broadcasted_iota(jnp.int32, probs.shape, 0)

    p1 = jnp.max(probs, axis=0, keepdims=True)
    idx1 = jnp.min(jnp.where(probs == p1, iota, _E), axis=0, keepdims=True)
    p2 = jnp.max(jnp.where(iota == idx1, -jnp.inf, probs), axis=0, keepdims=True)
    safe = (p2 / (p1 + 1e-9)) >= _RATIO

    mean_raw = jnp.mean(raw, axis=0, keepdims=True)
    var_raw = jnp.sum((raw - mean_raw) ** 2, axis=0, keepdims=True) / (_E - 1)
    mean_wm = jnp.mean(wm, axis=0, keepdims=True)
    var_wm = jnp.sum((wm - mean_wm) ** 2, axis=0, keepdims=True) / (_E - 1)
    scale = jnp.sqrt(var_raw) / (jnp.sqrt(var_wm) + 1e-9)

    combined = safe & (jnp.abs(wm) >= _DEAD)
    scores = jnp.where(combined, raw + _ALPHA * wm * scale, raw)

    idx_rows = []
    val_rows = []
    for _ in range(_K):
        mk = jnp.max(scores, axis=0, keepdims=True)
        idx = jnp.min(jnp.where(scores == mk, iota, _E), axis=0, keepdims=True)
        onehot = iota == idx
        val = jnp.sum(jnp.where(onehot, raw, 0.0), axis=0, keepdims=True)
        idx_rows.append(idx)
        val_rows.append(val)
        scores = jnp.where(onehot, -jnp.inf, scores)
    sel = jnp.concatenate(idx_rows, axis=0)
    vals = jnp.concatenate(val_rows, axis=0)

    mv = jnp.max(vals, axis=0, keepdims=True)
    ev = jnp.exp(vals - mv)
    weights_ref[...] = ev / jnp.sum(ev, axis=0, keepdims=True)
    experts_ref[...] = sel


@jax.jit
def kernel(hidden_states, W_gate, secret_projection):
    n_tok = _B * _S
    h2 = hidden_states.reshape(n_tok, _D)
    wct = jnp.concatenate([W_gate, secret_projection.T], axis=0)

    grid = (n_tok // _TOK_BLOCK,)
    weights_t, experts_t = pl.pallas_call(
        _router_body,
        grid=grid,
        in_specs=[
            pl.BlockSpec((_TOK_BLOCK, _D), lambda i: (i, 0)),
            pl.BlockSpec(memory_space=pl.ANY),
        ],
        out_specs=[
            pl.BlockSpec((_K, _TOK_BLOCK), lambda i: (0, i)),
            pl.BlockSpec((_K, _TOK_BLOCK), lambda i: (0, i)),
        ],
        out_shape=[
            jax.ShapeDtypeStruct((_K, n_tok), jnp.float32),
            jax.ShapeDtypeStruct((_K, n_tok), jnp.int32),
        ],
        scratch_shapes=[
            pltpu.VMEM((2 * _E, _D), jnp.float32),
            pltpu.SemaphoreType.DMA,
        ],
    )(h2, wct)
    return (
        weights_t.T.reshape(_B, _S, _K),
        experts_t.T.reshape(_B, _S, _K),
    )


# speedup vs baseline: 1.7117x; 1.0877x over previous
import jax
import jax.numpy as jnp
from jax.experimental import pallas as pl

_B, _S, _D, _E, _K = 4, 2048, 4096, 64, 8
_ALPHA = 0.1
_RATIO = 0.9
_DEAD = 0.01
_TOK_BLOCK = 1024


def _router_body(h_ref, wct_ref, weights_ref, experts_ref):
    h = h_ref[...]
    wct = wct_ref[...]
    logits = jax.lax.dot_general(
        wct, h, (((1,), (1,)), ((), ())), preferred_element_type=jnp.float32
    )
    raw = logits[:_E, :]
    wm = logits[_E:, :]

    m = jnp.max(raw, axis=0, keepdims=True)
    e = jnp.exp(raw - m)
    probs = e / jnp.sum(e, axis=0, keepdims=True)

    iota = jax.lax.broadcasted_iota(jnp.int32, probs.shape, 0)

    p1 = jnp.max(probs, axis=0, keepdims=True)
    idx1 = jnp.min(jnp.where(probs == p1, iota, _E), axis=0, keepdims=True)
    p2 = jnp.max(jnp.where(iota == idx1, -jnp.inf, probs), axis=0, keepdims=True)
    safe = (p2 / (p1 + 1e-9)) >= _RATIO

    mean_raw = jnp.mean(raw, axis=0, keepdims=True)
    var_raw = jnp.sum((raw - mean_raw) ** 2, axis=0, keepdims=True) / (_E - 1)
    mean_wm = jnp.mean(wm, axis=0, keepdims=True)
    var_wm = jnp.sum((wm - mean_wm) ** 2, axis=0, keepdims=True) / (_E - 1)
    scale = jnp.sqrt(var_raw) / (jnp.sqrt(var_wm) + 1e-9)

    combined = safe & (jnp.abs(wm) >= _DEAD)
    scores = jnp.where(combined, raw + _ALPHA * wm * scale, raw)

    idx_rows = []
    val_rows = []
    for _ in range(_K):
        mk = jnp.max(scores, axis=0, keepdims=True)
        idx = jnp.min(jnp.where(scores == mk, iota, _E), axis=0, keepdims=True)
        onehot = iota == idx
        val = jnp.sum(jnp.where(onehot, raw, 0.0), axis=0, keepdims=True)
        idx_rows.append(idx)
        val_rows.append(val)
        scores = jnp.where(onehot, -jnp.inf, scores)
    sel = jnp.concatenate(idx_rows, axis=0)
    vals = jnp.concatenate(val_rows, axis=0)

    mv = jnp.max(vals, axis=0, keepdims=True)
    ev = jnp.exp(vals - mv)
    weights_ref[...] = ev / jnp.sum(ev, axis=0, keepdims=True)
    experts_ref[...] = sel


@jax.jit
def kernel(hidden_states, W_gate, secret_projection):
    n_tok = _B * _S
    h2 = hidden_states.reshape(n_tok, _D)
    wct = jnp.concatenate([W_gate, secret_projection.T], axis=0)

    grid = (n_tok // _TOK_BLOCK,)
    weights_t, experts_t = pl.pallas_call(
        _router_body,
        grid=grid,
        in_specs=[
            pl.BlockSpec((_TOK_BLOCK, _D), lambda i: (i, 0)),
            pl.BlockSpec((2 * _E, _D), lambda i: (0, 0)),
        ],
        out_specs=[
            pl.BlockSpec((_K, _TOK_BLOCK), lambda i: (0, i)),
            pl.BlockSpec((_K, _TOK_BLOCK), lambda i: (0, i)),
        ],
        out_shape=[
            jax.ShapeDtypeStruct((_K, n_tok), jnp.float32),
            jax.ShapeDtypeStruct((_K, n_tok), jnp.int32),
        ],
    )(h2, wct)
    return (
        weights_t.T.reshape(_B, _S, _K),
        experts_t.T.reshape(_B, _S, _K),
    )


# speedup vs baseline: 1.7150x; 1.0019x over previous
import jax
import jax.numpy as jnp
from jax.experimental import pallas as pl
from jax.experimental.pallas import tpu as pltpu

_B, _S, _D, _E, _K = 4, 2048, 4096, 64, 8
_ALPHA = 0.1
_RATIO = 0.9
_DEAD = 0.01
_TOK_BLOCK = 1024


def _router_body(h_ref, wct_ref, weights_ref, experts_ref):
    h = h_ref[...]
    wct = wct_ref[...]
    logits = jax.lax.dot_general(
        wct, h, (((1,), (1,)), ((), ())), preferred_element_type=jnp.float32
    )
    raw = logits[:_E, :]
    wm = logits[_E:, :]

    m = jnp.max(raw, axis=0, keepdims=True)
    e = jnp.exp(raw - m)
    probs = e / jnp.sum(e, axis=0, keepdims=True)

    iota = jax.lax.broadcasted_iota(jnp.int32, probs.shape, 0)

    p1 = jnp.max(probs, axis=0, keepdims=True)
    idx1 = jnp.min(jnp.where(probs == p1, iota, _E), axis=0, keepdims=True)
    p2 = jnp.max(jnp.where(iota == idx1, -jnp.inf, probs), axis=0, keepdims=True)
    safe = (p2 / (p1 + 1e-9)) >= _RATIO

    mean_raw = jnp.mean(raw, axis=0, keepdims=True)
    var_raw = jnp.sum((raw - mean_raw) ** 2, axis=0, keepdims=True) / (_E - 1)
    mean_wm = jnp.mean(wm, axis=0, keepdims=True)
    var_wm = jnp.sum((wm - mean_wm) ** 2, axis=0, keepdims=True) / (_E - 1)
    scale = jnp.sqrt(var_raw) / (jnp.sqrt(var_wm) + 1e-9)

    combined = safe & (jnp.abs(wm) >= _DEAD)
    scores = jnp.where(combined, raw + _ALPHA * wm * scale, raw)

    idx_rows = []
    val_rows = []
    for _ in range(_K):
        mk = jnp.max(scores, axis=0, keepdims=True)
        idx = jnp.min(jnp.where(scores == mk, iota, _E), axis=0, keepdims=True)
        onehot = iota == idx
        val = jnp.sum(jnp.where(onehot, raw, 0.0), axis=0, keepdims=True)
        idx_rows.append(idx)
        val_rows.append(val)
        scores = jnp.where(onehot, -jnp.inf, scores)
    sel = jnp.concatenate(idx_rows, axis=0)
    vals = jnp.concatenate(val_rows, axis=0)

    mv = jnp.max(vals, axis=0, keepdims=True)
    ev = jnp.exp(vals - mv)
    weights_ref[...] = ev / jnp.sum(ev, axis=0, keepdims=True)
    experts_ref[...] = sel


@jax.jit
def kernel(hidden_states, W_gate, secret_projection):
    n_tok = _B * _S
    h2 = hidden_states.reshape(n_tok, _D)
    wct = jnp.concatenate([W_gate, secret_projection.T], axis=0)

    grid = (n_tok // _TOK_BLOCK,)
    weights_t, experts_t = pl.pallas_call(
        _router_body,
        grid=grid,
        in_specs=[
            pl.BlockSpec((_TOK_BLOCK, _D), lambda i: (i, 0)),
            pl.BlockSpec((2 * _E, _D), lambda i: (0, 0)),
        ],
        out_specs=[
            pl.BlockSpec((_K, _TOK_BLOCK), lambda i: (0, i)),
            pl.BlockSpec((_K, _TOK_BLOCK), lambda i: (0, i)),
        ],
        out_shape=[
            jax.ShapeDtypeStruct((_K, n_tok), jnp.float32),
            jax.ShapeDtypeStruct((_K, n_tok), jnp.int32),
        ],
        compiler_params=pltpu.CompilerParams(allow_input_fusion=[False, True]),
    )(h2, wct)
    return (
        weights_t.T.reshape(_B, _S, _K),
        experts_t.T.reshape(_B, _S, _K),
    )
